# Initial kernel scaffold; baseline (speedup 1.0000x reference)
import functools

import jax
import jax.numpy as jnp
from jax import lax
from jax.experimental import pallas as pl
from jax.experimental.pallas import tpu as pltpu
from jax.experimental.pallas import tpu_sc as plsc

B, N, F = 8, 16384, 128
HALF = N // 2
FH = F // 2
NSUB = 16
ROWS_PER_SUB = N // NSUB
CHUNK = 128
NCHUNK = ROWS_PER_SUB // CHUNK


def _g_matmul_body(feats_ref, w2_ref, g_ref):
    x = feats_ref[0]
    y = jnp.dot(x, w2_ref[...], preferred_element_type=jnp.float32)
    g_ref[0, 0] = y[:, :FH]
    g_ref[1, 0] = y[:, FH:]


def _g_matmul(feats, w2):
    return pl.pallas_call(
        _g_matmul_body,
        grid=(B,),
        in_specs=[
            pl.BlockSpec((1, HALF, F), lambda b: (b, 0, 0)),
            pl.BlockSpec((F, F), lambda b: (0, 0)),
        ],
        out_specs=pl.BlockSpec((2, 1, HALF, FH), lambda b: (0, b, 0, 0)),
        out_shape=jax.ShapeDtypeStruct((2, B, HALF, FH), jnp.float32),
    )(feats, w2)


def _sc_scatter(g, src_idx, ups_idx, zeros_tile):
    mesh = plsc.VectorSubcoreMesh(core_axis_name="c", subcore_axis_name="s")

    @functools.partial(
        pl.kernel,
        out_type=jax.ShapeDtypeStruct((2, B, N, FH), jnp.float32),
        mesh=mesh,
        scratch_types=[
            pltpu.VMEM((NCHUNK, CHUNK), jnp.int32),
            pltpu.VMEM((NCHUNK, CHUNK), jnp.int32),
            pltpu.VMEM((CHUNK, FH), jnp.float32),
            pltpu.VMEM((CHUNK, FH), jnp.float32),
            pltpu.VMEM_SHARED((N, FH), jnp.float32),
            pltpu.SemaphoreType.DMA,
        ],
    )
    def k(g_hbm, src_hbm, ups_hbm, z_hbm, out_hbm, sidx, uidx, rows, ztile, acc, sem):
        c = lax.axis_index("c")
        s = lax.axis_index("s")
        pltpu.sync_copy(z_hbm, ztile)

        @pl.loop(0, B)
        def _batch(b):
            @pl.loop(0, NCHUNK)
            def _z(j):
                pltpu.sync_copy(ztile, acc.at[pl.ds(s * ROWS_PER_SUB + j * CHUNK, CHUNK)])

            plsc.subcore_barrier()

            pltpu.sync_copy(src_hbm.at[b].at[s], sidx)
            pltpu.sync_copy(ups_hbm.at[b].at[s], uidx)

            @pl.loop(0, NCHUNK)
            def _c(j):
                pltpu.async_copy(g_hbm.at[c].at[b].at[sidx.at[j]], rows, sem).wait()
                pltpu.sync_copy(rows, acc.at[uidx.at[j]], add=True)

            plsc.subcore_barrier()

            pltpu.sync_copy(
                acc.at[pl.ds(s * ROWS_PER_SUB, ROWS_PER_SUB)],
                out_hbm.at[c].at[b].at[pl.ds(s * ROWS_PER_SUB, ROWS_PER_SUB)],
            )
            plsc.subcore_barrier()

    return k(g, src_idx, ups_idx, zeros_tile)


ROW_BLK = 1024


def _final_body(res_ref, acc_ref, w1_ref, bias_ref, out_ref):
    r = res_ref[0]
    a = jnp.concatenate([acc_ref[0, 0], acc_ref[1, 0]], axis=-1)
    y = jnp.dot(r, w1_ref[...], preferred_element_type=jnp.float32)
    out_ref[0] = jnp.maximum(y + a + bias_ref[...], 0.0)


def _final(res, acc, w1, bias):
    return pl.pallas_call(
        _final_body,
        grid=(B, N // ROW_BLK),
        in_specs=[
            pl.BlockSpec((1, ROW_BLK, F), lambda b, i: (b, i, 0)),
            pl.BlockSpec((2, 1, ROW_BLK, FH), lambda b, i: (0, b, i, 0)),
            pl.BlockSpec((F, F), lambda b, i: (0, 0)),
            pl.BlockSpec((1, F), lambda b, i: (0, 0)),
        ],
        out_specs=pl.BlockSpec((1, ROW_BLK, F), lambda b, i: (b, i, 0)),
        out_shape=jax.ShapeDtypeStruct((B, N, F), jnp.float32),
    )(res, acc, w1, bias)


def kernel(feats, interpolate_idx, upsample_idx, res, W, b):
    w1 = W[:F, :]
    w2 = W[F:, :]
    iota = jnp.broadcast_to(jnp.arange(HALF, dtype=jnp.int32), (B, HALF))
    src_idx = jnp.concatenate([iota, interpolate_idx.astype(jnp.int32)], axis=1)
    src_r = src_idx.reshape(B, NSUB, NCHUNK, CHUNK)
    ups_r = upsample_idx.astype(jnp.int32).reshape(B, NSUB, NCHUNK, CHUNK)
    zeros_tile = jnp.zeros((CHUNK, FH), jnp.float32)

    g = _g_matmul(feats, w2)
    acc = _sc_scatter(g, src_r, ups_r, zeros_tile)
    return _final(res, acc, w1, b.reshape(1, F))


# speedup vs baseline: 1.7937x; 1.7937x over previous
import functools

import jax
import jax.numpy as jnp
from jax import lax
from jax.experimental import pallas as pl
from jax.experimental.pallas import tpu as pltpu
from jax.experimental.pallas import tpu_sc as plsc

B, N, F = 8, 16384, 128
HALF = N // 2
NSUB = 16
ROWS_PER_SUB = N // NSUB
CHUNK = 128
NCHUNK = ROWS_PER_SUB // CHUNK
TRASH = 128
ACC_ROWS = HALF + TRASH
OUT_PER_SUB = HALF // NSUB


def _g_matmul_body(feats_ref, w2_ref, g_ref):
    x = feats_ref[0]
    g_ref[0] = jnp.dot(x, w2_ref[...], preferred_element_type=jnp.float32)


def _g_matmul(feats, w2):
    return pl.pallas_call(
        _g_matmul_body,
        grid=(B,),
        in_specs=[
            pl.BlockSpec((1, HALF, F), lambda b: (b, 0, 0)),
            pl.BlockSpec((F, F), lambda b: (0, 0)),
        ],
        out_specs=pl.BlockSpec((1, HALF, F), lambda b: (b, 0, 0)),
        out_shape=jax.ShapeDtypeStruct((B, HALF, F), jnp.float32),
    )(feats, w2)


def _sc_scatter(g, src_idx, ups_idx):
    mesh = plsc.VectorSubcoreMesh(core_axis_name="c", subcore_axis_name="s")

    @functools.partial(
        pl.kernel,
        out_type=jax.ShapeDtypeStruct((B, N, F), jnp.float32),
        mesh=mesh,
        scratch_types=[
            pltpu.VMEM((NCHUNK, CHUNK), jnp.int32),
            pltpu.VMEM((NCHUNK, CHUNK), jnp.int32),
            pltpu.VMEM((CHUNK, F), jnp.float32),
            pltpu.VMEM((CHUNK, F), jnp.float32),
            pltpu.VMEM_SHARED((ACC_ROWS, F), jnp.float32),
            pltpu.SemaphoreType.DMA,
        ],
    )
    def k(g_hbm, src_hbm, ups_hbm, out_hbm, sidx, uidx, rows, ztile, acc, sem):
        c = lax.axis_index("c")
        s = lax.axis_index("s")

        @pl.loop(0, CHUNK)
        def _zrow(i):
            @pl.loop(0, F, step=16)
            def _zcol(j):
                ztile[i, pl.ds(j, 16)] = jnp.zeros((16,), jnp.float32)

        @pl.loop(0, B)
        def _batch(b):
            @pl.loop(0, OUT_PER_SUB, step=CHUNK)
            def _z(r):
                pltpu.sync_copy(ztile, acc.at[pl.ds(s * OUT_PER_SUB + r, CHUNK)])

            @pl.when(s == NSUB - 1)
            def _ztrash():
                pltpu.sync_copy(ztile, acc.at[pl.ds(HALF, TRASH)])

            plsc.subcore_barrier()

            pltpu.sync_copy(src_hbm.at[b].at[s], sidx)
            pltpu.sync_copy(ups_hbm.at[c].at[b].at[s], uidx)

            @pl.loop(0, NCHUNK)
            def _c(j):
                pltpu.async_copy(g_hbm.at[b].at[sidx.at[j]], rows, sem).wait()
                pltpu.sync_copy(rows, acc.at[uidx.at[j]], add=True)

            plsc.subcore_barrier()

            pltpu.sync_copy(
                acc.at[pl.ds(s * OUT_PER_SUB, OUT_PER_SUB)],
                out_hbm.at[b].at[pl.ds(c * HALF + s * OUT_PER_SUB, OUT_PER_SUB)],
            )
            plsc.subcore_barrier()

    return k(g, src_idx, ups_idx)


ROW_BLK = 1024


def _final_body(res_ref, acc_ref, w1_ref, bias_ref, out_ref):
    r = res_ref[0]
    y = jnp.dot(r, w1_ref[...], preferred_element_type=jnp.float32)
    out_ref[0] = jnp.maximum(y + acc_ref[0] + bias_ref[...], 0.0)


def _final(res, acc, w1, bias):
    return pl.pallas_call(
        _final_body,
        grid=(B, N // ROW_BLK),
        in_specs=[
            pl.BlockSpec((1, ROW_BLK, F), lambda b, i: (b, i, 0)),
            pl.BlockSpec((1, ROW_BLK, F), lambda b, i: (b, i, 0)),
            pl.BlockSpec((F, F), lambda b, i: (0, 0)),
            pl.BlockSpec((1, F), lambda b, i: (0, 0)),
        ],
        out_specs=pl.BlockSpec((1, ROW_BLK, F), lambda b, i: (b, i, 0)),
        out_shape=jax.ShapeDtypeStruct((B, N, F), jnp.float32),
    )(res, acc, w1, bias)


def kernel(feats, interpolate_idx, upsample_idx, res, W, b):
    w1 = W[:F, :]
    w2 = W[F:, :]
    iota = jnp.broadcast_to(jnp.arange(HALF, dtype=jnp.int32), (B, HALF))
    src_idx = jnp.concatenate([iota, interpolate_idx.astype(jnp.int32)], axis=1)
    src_r = src_idx.reshape(B, NSUB, NCHUNK, CHUNK)

    ups = upsample_idx.astype(jnp.int32)
    trash = HALF + (jnp.arange(N, dtype=jnp.int32) % TRASH)[None, :]
    ups_lo = jnp.where(ups < HALF, ups, trash)
    ups_hi = jnp.where(ups >= HALF, ups - HALF, trash)
    ups_r = jnp.stack([ups_lo, ups_hi]).reshape(2, B, NSUB, NCHUNK, CHUNK)

    g = _g_matmul(feats, w2)
    acc = _sc_scatter(g, src_r, ups_r)
    return _final(res, acc, w1, b.reshape(1, F))
